# Initial kernel scaffold; baseline (speedup 1.0000x reference)
import jax
import jax.numpy as jnp
from jax.experimental import pallas as pl


def kernel(ids, embeddings):
    raise NotImplementedError("write your pallas kernel here")


# speedup vs baseline: 1.1037x; 1.1037x over previous
import functools

import jax
import jax.numpy as jnp
from jax import lax
from jax.experimental import pallas as pl
from jax.experimental.pallas import tpu as pltpu
from jax.experimental.pallas import tpu_sc as plsc

N_ROWS = 16384 * 50
EMBED_D = 32
IDX_W = 128
NUM_WORKERS = 32
ROWS_PER_W = N_ROWS // NUM_WORKERS
IDXROWS_PER_W = ROWS_PER_W // IDX_W
CHUNK = 1024
GATHERS_PER_CHUNK = CHUNK // IDX_W
NCHUNKS = ROWS_PER_W // CHUNK


def _gather_body(idx_hbm, table_hbm, out_hbm, idx_v, rows_v, sem):
    wid = lax.axis_index("s") * 2 + lax.axis_index("c")
    pltpu.sync_copy(idx_hbm.at[pl.ds(wid * IDXROWS_PER_W, IDXROWS_PER_W)], idx_v)
    out_base = wid * ROWS_PER_W

    def chunk_step(c, _):
        descs = []
        for g in range(GATHERS_PER_CHUNK):
            descs.append(
                pltpu.async_copy(
                    table_hbm.at[idx_v.at[c * GATHERS_PER_CHUNK + g]],
                    rows_v.at[pl.ds(g * IDX_W, IDX_W)],
                    sem,
                )
            )
        for d in descs:
            d.wait()
        pltpu.sync_copy(rows_v, out_hbm.at[pl.ds(out_base + c * CHUNK, CHUNK)])
        return 0

    lax.fori_loop(0, NCHUNKS, chunk_step, 0)


_gather = functools.partial(
    pl.kernel,
    mesh=plsc.VectorSubcoreMesh(core_axis_name="c", subcore_axis_name="s"),
    out_type=jax.ShapeDtypeStruct((N_ROWS, EMBED_D), jnp.float32),
    scratch_types=[
        pltpu.VMEM((IDXROWS_PER_W, IDX_W), jnp.int32),
        pltpu.VMEM((CHUNK, EMBED_D), jnp.float32),
        pltpu.SemaphoreType.DMA,
    ],
    compiler_params=pltpu.CompilerParams(use_tc_tiling_on_sc=False),
)(_gather_body)


def kernel(ids, embeddings):
    idx2d = ids.reshape(N_ROWS // IDX_W, IDX_W)
    out = _gather(idx2d, embeddings)
    return out.reshape(*ids.shape, EMBED_D)


# speedup vs baseline: 1.1099x; 1.0056x over previous
import functools

import jax
import jax.numpy as jnp
from jax import lax
from jax.experimental import pallas as pl
from jax.experimental.pallas import tpu as pltpu
from jax.experimental.pallas import tpu_sc as plsc

N_ROWS = 16384 * 50
EMBED_D = 32
IDX_W = 128
NUM_WORKERS = 32
ROWS_PER_W = N_ROWS // NUM_WORKERS
IDXROWS_PER_W = ROWS_PER_W // IDX_W
CHUNK = 1280
GPC = CHUNK // IDX_W
NCHUNKS = ROWS_PER_W // CHUNK


def _gather_body(idx_hbm, table_hbm, out_hbm, idx_v, rows_a, rows_b, sem_ga,
                 sem_gb, sem_oa, sem_ob):
    wid = lax.axis_index("s") * 2 + lax.axis_index("c")
    pltpu.sync_copy(idx_hbm.at[pl.ds(wid * IDXROWS_PER_W, IDXROWS_PER_W)], idx_v)
    out_base = wid * ROWS_PER_W

    def fire_gathers(c, buf, sem):
        for g in range(GPC):
            pltpu.async_copy(
                table_hbm.at[idx_v.at[c * GPC + g]],
                buf.at[pl.ds(g * IDX_W, IDX_W)],
                sem,
            )

    def drain_gathers(buf, sem):
        pltpu.make_async_copy(out_hbm.at[pl.ds(0, CHUNK)], buf, sem).wait()

    def fire_out(c, buf, sem):
        pltpu.async_copy(buf, out_hbm.at[pl.ds(out_base + c * CHUNK, CHUNK)], sem)

    def drain_out(c, buf, sem):
        pltpu.make_async_copy(
            buf, out_hbm.at[pl.ds(out_base + c * CHUNK, CHUNK)], sem).wait()

    fire_gathers(0, rows_a, sem_ga)
    fire_gathers(1, rows_b, sem_gb)

    def group_step(g, _):
        c = 2 * g
        drain_gathers(rows_a, sem_ga)
        fire_out(c, rows_a, sem_oa)
        drain_gathers(rows_b, sem_gb)
        fire_out(c + 1, rows_b, sem_ob)
        drain_out(c, rows_a, sem_oa)
        fire_gathers(c + 2, rows_a, sem_ga)
        drain_out(c + 1, rows_b, sem_ob)
        fire_gathers(c + 3, rows_b, sem_gb)
        return 0

    lax.fori_loop(0, NCHUNKS // 2 - 1, group_step, 0)

    c = NCHUNKS - 2
    drain_gathers(rows_a, sem_ga)
    fire_out(c, rows_a, sem_oa)
    drain_gathers(rows_b, sem_gb)
    fire_out(c + 1, rows_b, sem_ob)
    drain_out(c, rows_a, sem_oa)
    drain_out(c + 1, rows_b, sem_ob)


_gather = functools.partial(
    pl.kernel,
    mesh=plsc.VectorSubcoreMesh(core_axis_name="c", subcore_axis_name="s"),
    out_type=jax.ShapeDtypeStruct((N_ROWS, EMBED_D), jnp.float32),
    scratch_types=[
        pltpu.VMEM((IDXROWS_PER_W, IDX_W), jnp.int32),
        pltpu.VMEM((CHUNK, EMBED_D), jnp.float32),
        pltpu.VMEM((CHUNK, EMBED_D), jnp.float32),
        pltpu.SemaphoreType.DMA,
        pltpu.SemaphoreType.DMA,
        pltpu.SemaphoreType.DMA,
        pltpu.SemaphoreType.DMA,
    ],
    compiler_params=pltpu.CompilerParams(use_tc_tiling_on_sc=False),
)(_gather_body)


def kernel(ids, embeddings):
    idx2d = ids.reshape(N_ROWS // IDX_W, IDX_W)
    out = _gather(idx2d, embeddings)
    return out.reshape(*ids.shape, EMBED_D)


# speedup vs baseline: 1.7821x; 1.6057x over previous
import functools

import jax
import jax.numpy as jnp
from jax import lax
from jax.experimental import pallas as pl
from jax.experimental.pallas import tpu as pltpu
from jax.experimental.pallas import tpu_sc as plsc

N_IDROWS = 16384
ROW_W = 50
EMBED_D = 32
NUM_WORKERS = 32
IDROWS_PER_W = N_IDROWS // NUM_WORKERS
CHUNK = 8
NCHUNKS = IDROWS_PER_W // CHUNK


def _gather_body(idx_hbm, table_hbm, out_hbm, idx_v, rows_a, rows_b, sem_ga,
                 sem_gb, sem_oa, sem_ob):
    wid = lax.axis_index("s") * 2 + lax.axis_index("c")
    row0 = wid * IDROWS_PER_W
    pltpu.sync_copy(idx_hbm.at[pl.ds(row0, IDROWS_PER_W)], idx_v)

    def fire_gathers(c, buf, sem):
        for g in range(CHUNK):
            pltpu.async_copy(
                table_hbm.at[idx_v.at[c * CHUNK + g]],
                buf.at[g],
                sem,
            )

    def drain_gathers(buf, sem):
        pltpu.make_async_copy(out_hbm.at[pl.ds(0, CHUNK)], buf, sem).wait()

    def fire_out(c, buf, sem):
        pltpu.async_copy(buf, out_hbm.at[pl.ds(row0 + c * CHUNK, CHUNK)], sem)

    def drain_out(c, buf, sem):
        pltpu.make_async_copy(
            buf, out_hbm.at[pl.ds(row0 + c * CHUNK, CHUNK)], sem).wait()

    fire_gathers(0, rows_a, sem_ga)
    fire_gathers(1, rows_b, sem_gb)

    def group_step(g, _):
        c = 2 * g
        drain_gathers(rows_a, sem_ga)
        fire_out(c, rows_a, sem_oa)
        drain_gathers(rows_b, sem_gb)
        fire_out(c + 1, rows_b, sem_ob)
        drain_out(c, rows_a, sem_oa)
        fire_gathers(c + 2, rows_a, sem_ga)
        drain_out(c + 1, rows_b, sem_ob)
        fire_gathers(c + 3, rows_b, sem_gb)
        return 0

    lax.fori_loop(0, NCHUNKS // 2 - 1, group_step, 0)

    c = NCHUNKS - 2
    drain_gathers(rows_a, sem_ga)
    fire_out(c, rows_a, sem_oa)
    drain_gathers(rows_b, sem_gb)
    fire_out(c + 1, rows_b, sem_ob)
    drain_out(c, rows_a, sem_oa)
    drain_out(c + 1, rows_b, sem_ob)


_gather = functools.partial(
    pl.kernel,
    mesh=plsc.VectorSubcoreMesh(core_axis_name="c", subcore_axis_name="s"),
    out_type=jax.ShapeDtypeStruct((N_IDROWS, ROW_W, EMBED_D), jnp.float32),
    scratch_types=[
        pltpu.VMEM((IDROWS_PER_W, ROW_W), jnp.int32),
        pltpu.VMEM((CHUNK, ROW_W, EMBED_D), jnp.float32),
        pltpu.VMEM((CHUNK, ROW_W, EMBED_D), jnp.float32),
        pltpu.SemaphoreType.DMA,
        pltpu.SemaphoreType.DMA,
        pltpu.SemaphoreType.DMA,
        pltpu.SemaphoreType.DMA,
    ],
    compiler_params=pltpu.CompilerParams(use_tc_tiling_on_sc=False),
)(_gather_body)


def kernel(ids, embeddings):
    return _gather(ids, embeddings)
